# baseline (speedup 1.0000x reference)
import jax
import jax.numpy as jnp
from jax.experimental import pallas as pl
from jax.experimental.pallas import tpu as pltpu

N_EXPERTS = 64
N_GROUPS = 8
GROUP_SIZE = N_EXPERTS // N_GROUPS
TOPK_GROUPS = 4
TOPK = 8
SCALING_FACTOR = 2.5


def _router_body(h_ref, w_ref, rb_ref, sb_ref, idx_ref, wts_ref):
    bt = h_ref.shape[0]
    h = h_ref[...]
    w = w_ref[...]
    logits = jax.lax.dot_general(
        w, h, (((1,), (1,)), ((), ())), preferred_element_type=jnp.float32
    )
    logits = logits + rb_ref[...]
    s = jax.nn.sigmoid(logits) + sb_ref[...]

    m = jnp.maximum(s[0:32, :], s[32:64, :])
    m2 = jnp.minimum(s[0:32, :], s[32:64, :])
    for half in (16, 8):
        a, b = m[:half, :], m[half : 2 * half, :]
        sa, sb = m2[:half, :], m2[half : 2 * half, :]
        m2 = jnp.maximum(jnp.minimum(a, b), jnp.maximum(sa, sb))
        m = jnp.maximum(a, b)
    gw = m + m2

    ri8 = jax.lax.broadcasted_iota(jnp.int32, (N_GROUPS, bt), 0)
    selmask8 = jnp.zeros((N_GROUPS, bt), dtype=jnp.bool_)
    for _ in range(TOPK_GROUPS):
        mg = jnp.max(gw, axis=0, keepdims=True)
        gidx = jnp.min(
            jnp.where(gw == mg, ri8, N_GROUPS), axis=0, keepdims=True
        )
        eq = ri8 == gidx
        selmask8 = jnp.logical_or(selmask8, eq)
        gw = jnp.where(eq, -1.0, gw)

    bigmask = jnp.concatenate([selmask8] * N_GROUPS, axis=0)
    masked = jnp.where(bigmask, s, 0.0)

    ri64 = jax.lax.broadcasted_iota(jnp.int32, (N_EXPERTS, bt), 0)
    eid = (ri64 & 7) * 8 + (ri64 >> 3)
    idx_rows, val_rows = [], []
    for _ in range(TOPK):
        mv = jnp.max(masked, axis=0, keepdims=True)
        idx = jnp.min(
            jnp.where(masked == mv, eid, N_EXPERTS), axis=0, keepdims=True
        )
        idx_rows.append(idx)
        val_rows.append(mv)
        masked = jnp.where(eid == idx, -1.0, masked)
    vals = jnp.concatenate(val_rows, axis=0)
    idxs = jnp.concatenate(idx_rows, axis=0)
    wsum = jnp.sum(vals, axis=0, keepdims=True) + 1e-20
    idx_ref[...] = idxs
    wts_ref[...] = vals / wsum * SCALING_FACTOR


def kernel(hidden_tensor, weight, router_bias, scores_bias):
    t, d = hidden_tensor.shape
    bt = 4096 if t % 4096 == 0 else t
    grid = t // bt
    perm = jnp.arange(N_EXPERTS).reshape(N_GROUPS, GROUP_SIZE).T.reshape(-1)
    w_p = weight[perm]
    rb = router_bias[perm].reshape(N_EXPERTS, 1)
    sb = scores_bias[perm].reshape(N_EXPERTS, 1)
    idx_t, wts_t = pl.pallas_call(
        _router_body,
        grid=(grid,),
        in_specs=[
            pl.BlockSpec((bt, d), lambda i: (i, 0)),
            pl.BlockSpec((N_EXPERTS, d), lambda i: (0, 0)),
            pl.BlockSpec((N_EXPERTS, 1), lambda i: (0, 0)),
            pl.BlockSpec((N_EXPERTS, 1), lambda i: (0, 0)),
        ],
        out_specs=[
            pl.BlockSpec((TOPK, bt), lambda i: (0, i)),
            pl.BlockSpec((TOPK, bt), lambda i: (0, i)),
        ],
        out_shape=[
            jax.ShapeDtypeStruct((TOPK, t), jnp.int32),
            jax.ShapeDtypeStruct((TOPK, t), jnp.float32),
        ],
        compiler_params=pltpu.CompilerParams(
            dimension_semantics=("parallel",),
        ),
    )(hidden_tensor, w_p, rb, sb)
    return idx_t.T, wts_t.T
